# baseline (speedup 1.0000x reference)
import jax
import jax.numpy as jnp
from jax import lax
from jax.experimental import pallas as pl
from jax.experimental.pallas import tpu as pltpu
from jax.experimental.pallas import tpu_sc as plsc

DIM = 128
GLOVE_DIM = 300
GLOVE_PAD = 304
OUT_DIM = DIM + GLOVE_DIM

NC = 2
NS = 16
NW = NC * NS
LANES = 16

CHUNK = 128
TAIL = GLOVE_DIM - 296


def _tanh16(v):
    a = jnp.abs(v)
    t = jnp.exp(a * (-2.0))
    r = (1.0 - t) / (1.0 + t)
    return jnp.where(v < 0.0, -r, r)


def _sc_body(x_hbm, glove_hbm, table_hbm, out_hbm,
             idx_v, g_v, t_v, asm_v, sem_g, sem_t):
    wid = lax.axis_index("s") * NC + lax.axis_index("c")
    n_total = x_hbm.shape[0]
    per_w = n_total // NW
    n_chunks = per_w // CHUNK
    base_w = wid * per_w

    def chunk_body(i, carry):
        base = base_w + i * CHUNK
        pltpu.sync_copy(x_hbm.at[pl.ds(base, CHUNK)], idx_v)
        g_cp = pltpu.async_copy(glove_hbm.at[idx_v], g_v, sem_g)
        t_cp = pltpu.async_copy(table_hbm.at[idx_v], t_v, sem_t)
        t_cp.wait()
        g_cp.wait()

        def row_body(r, carry2):
            for j in range(DIM // LANES):
                sl = pl.ds(j * LANES, LANES)
                asm_v[r, sl] = _tanh16(t_v[r, sl])
            for j in range(288 // LANES):
                asm_v[r, pl.ds(DIM + j * LANES, LANES)] = \
                    g_v[r, pl.ds(j * LANES, LANES)]
            return carry2

        lax.fori_loop(0, CHUNK, row_body, 0, unroll=2)

        def tail_body(v, carry2):
            q = jax.lax.iota(jnp.int32, LANES) + v * LANES
            r = q // 12
            c = q - r * 12
            vals = plsc.load_gather(g_v, [r, c + 288])
            plsc.store_scatter(asm_v, [r, c + (DIM + 288)], vals)
            return carry2

        lax.fori_loop(0, CHUNK * 12 // LANES, tail_body, 0, unroll=2)
        pltpu.sync_copy(asm_v, out_hbm.at[pl.ds(base, CHUNK)])
        return carry

    lax.fori_loop(0, n_chunks, chunk_body, 0)


@jax.jit
def _glove_fused(x_flat, glove_padded, table):
    n = x_flat.shape[0]
    mesh = plsc.VectorSubcoreMesh(
        core_axis_name="c", subcore_axis_name="s",
        num_cores=NC, num_subcores=NS)
    return pl.kernel(
        _sc_body,
        out_type=jax.ShapeDtypeStruct((n, OUT_DIM), jnp.float32),
        mesh=mesh,
        scratch_types=[
            pltpu.VMEM((CHUNK,), jnp.int32),
            pltpu.VMEM((CHUNK, GLOVE_PAD), jnp.float32),
            pltpu.VMEM((CHUNK, DIM), jnp.float32),
            pltpu.VMEM((CHUNK, OUT_DIM), jnp.float32),
            pltpu.SemaphoreType.DMA,
            pltpu.SemaphoreType.DMA,
        ],
        compiler_params=pltpu.CompilerParams(
            use_tc_tiling_on_sc=False, needs_layout_passes=False),
    )(x_flat, glove_padded, table)


def kernel(x, glove_table, table):
    b, l = x.shape
    glove_padded = jnp.pad(glove_table, ((0, 0), (0, GLOVE_PAD - GLOVE_DIM)))
    out = _glove_fused(x.reshape(b * l), glove_padded, table)
    return out.reshape(b, l, OUT_DIM)


# speedup vs baseline: 1.1856x; 1.1856x over previous
import jax
import jax.numpy as jnp
from jax import lax
from jax.experimental import pallas as pl
from jax.experimental.pallas import tpu as pltpu
from jax.experimental.pallas import tpu_sc as plsc

DIM = 128
GLOVE_DIM = 300
GLOVE_PAD = 304
OUT_DIM = DIM + GLOVE_DIM
GMAIN = 288
GTAIL = GLOVE_DIM - GMAIN

NC = 2
NS = 16
NW = NC * NS
LANES = 16

CHUNK = 64


def _tanh16(v):
    a = jnp.abs(v)
    t = jnp.exp(a * (-2.0))
    r = (1.0 - t) / (1.0 + t)
    return jnp.where(v < 0.0, -r, r)


def _sc_body(x_hbm, glove_hbm, table_hbm, out_hbm,
             idx_v, g_v, t_v, asm_v, sem_g, sem_t, sem_w):
    wid = lax.axis_index("s") * NC + lax.axis_index("c")
    n_total = x_hbm.shape[0]
    per_w = n_total // NW
    n_chunks = per_w // CHUNK
    n2 = n_chunks // 2
    base_w = wid * per_w

    def start_gathers(c, s):
        base = base_w + c * CHUNK
        pltpu.sync_copy(x_hbm.at[pl.ds(base, CHUNK)], idx_v.at[s])
        pltpu.async_copy(glove_hbm.at[idx_v.at[s]], g_v.at[s], sem_g.at[s])
        pltpu.async_copy(table_hbm.at[idx_v.at[s]], t_v.at[s], sem_t.at[s])

    def wait_gathers(s):
        pltpu.make_async_copy(glove_hbm.at[idx_v.at[s]], g_v.at[s],
                              sem_g.at[s]).wait()
        pltpu.make_async_copy(table_hbm.at[idx_v.at[s]], t_v.at[s],
                              sem_t.at[s]).wait()

    def drain_write(c, s):
        base = base_w + c * CHUNK
        pltpu.make_async_copy(asm_v.at[s], out_hbm.at[pl.ds(base, CHUNK)],
                              sem_w.at[s]).wait()

    def assemble_and_write(c, s):
        def row_body(r, carry2):
            for j in range(DIM // LANES):
                sl = pl.ds(j * LANES, LANES)
                asm_v[s, r, sl] = _tanh16(t_v[s, r, sl])
            for j in range(GMAIN // LANES):
                asm_v[s, r, pl.ds(DIM + j * LANES, LANES)] = \
                    g_v[s, r, pl.ds(j * LANES, LANES)]
            return carry2

        lax.fori_loop(0, CHUNK, row_body, 0, unroll=4)

        def tail_body(v, carry2):
            q = jax.lax.iota(jnp.int32, LANES) + v * LANES
            r = q // GTAIL
            col = q - r * GTAIL
            vals = plsc.load_gather(g_v.at[s], [r, col + GMAIN])
            plsc.store_scatter(asm_v.at[s], [r, col + (DIM + GMAIN)], vals)
            return carry2

        lax.fori_loop(0, CHUNK * GTAIL // LANES, tail_body, 0, unroll=4)
        base = base_w + c * CHUNK
        pltpu.async_copy(asm_v.at[s], out_hbm.at[pl.ds(base, CHUNK)],
                         sem_w.at[s])

    start_gathers(0, 0)

    def body2(k, carry):
        c0 = 2 * k
        c1 = 2 * k + 1
        start_gathers(c1, 1)
        wait_gathers(0)

        @pl.when(k > 0)
        def _():
            drain_write(c0 - 2, 0)

        assemble_and_write(c0, 0)

        @pl.when(k < n2 - 1)
        def _():
            start_gathers(c0 + 2, 0)

        wait_gathers(1)

        @pl.when(k > 0)
        def _():
            drain_write(c1 - 2, 1)

        assemble_and_write(c1, 1)
        return carry

    lax.fori_loop(0, n2, body2, 0)
    drain_write(n_chunks - 2, 0)
    drain_write(n_chunks - 1, 1)


def _pad_body(g_ref, o_ref):
    o_ref[:, :GLOVE_DIM] = g_ref[...]
    o_ref[:, GLOVE_DIM:] = jnp.zeros(
        (o_ref.shape[0], GLOVE_PAD - GLOVE_DIM), jnp.float32)


def _pad_glove(glove_table):
    v = glove_table.shape[0]
    rows = 2000
    return pl.pallas_call(
        _pad_body,
        grid=(v // rows,),
        in_specs=[pl.BlockSpec((rows, GLOVE_DIM), lambda i: (i, 0))],
        out_specs=pl.BlockSpec((rows, GLOVE_PAD), lambda i: (i, 0)),
        out_shape=jax.ShapeDtypeStruct((v, GLOVE_PAD), jnp.float32),
    )(glove_table)


@jax.jit
def _glove_fused(x_flat, glove_table, table):
    n = x_flat.shape[0]
    glove_padded = _pad_glove(glove_table)
    mesh = plsc.VectorSubcoreMesh(
        core_axis_name="c", subcore_axis_name="s",
        num_cores=NC, num_subcores=NS)
    return pl.kernel(
        _sc_body,
        out_type=jax.ShapeDtypeStruct((n, OUT_DIM), jnp.float32),
        mesh=mesh,
        scratch_types=[
            pltpu.VMEM((2, CHUNK), jnp.int32),
            pltpu.VMEM((2, CHUNK, GLOVE_PAD), jnp.float32),
            pltpu.VMEM((2, CHUNK, DIM), jnp.float32),
            pltpu.VMEM((2, CHUNK, OUT_DIM), jnp.float32),
            pltpu.SemaphoreType.DMA((2,)),
            pltpu.SemaphoreType.DMA((2,)),
            pltpu.SemaphoreType.DMA((2,)),
        ],
        compiler_params=pltpu.CompilerParams(
            use_tc_tiling_on_sc=False, needs_layout_passes=False),
    )(x_flat, glove_padded, table)


def kernel(x, glove_table, table):
    b, l = x.shape
    out = _glove_fused(x.reshape(b * l), glove_table, table)
    return out.reshape(b, l, OUT_DIM)


# speedup vs baseline: 2.5828x; 2.1785x over previous
import jax
import jax.numpy as jnp
from jax import lax
from jax.experimental import pallas as pl
from jax.experimental.pallas import tpu as pltpu
from jax.experimental.pallas import tpu_sc as plsc

DIM = 128
GLOVE_DIM = 300
OUT_DIM = DIM + GLOVE_DIM
COMB_DIM = 432
CMAIN = 416
CTAIL = OUT_DIM - CMAIN

NC = 2
NS = 16
NW = NC * NS
LANES = 16

CB = 2
L = 50
CL = CB * L


def _sc_body(x_hbm, comb_hbm, out_hbm,
             idx2_a, idx2_b, idx_a, idx_b, comb_a, comb_b, asm_v,
             sem_a, sem_b, sem_w):
    wid = lax.axis_index("s") * NC + lax.axis_index("c")
    nb_total = x_hbm.shape[0]
    per_w = nb_total // NW
    n_chunks = per_w // CB
    n2 = n_chunks // 2
    base_w = wid * per_w

    def start_gather(c, idx2_v, idx_v, comb_v, sem):
        b0 = base_w + c * CB
        pltpu.sync_copy(x_hbm.at[pl.ds(b0, CB)], idx2_v)

        def flat_body(m, carry2):
            q = jax.lax.iota(jnp.int32, LANES) + m * LANES
            msk = q < CL
            qc = jnp.minimum(q, CL - 1)
            bb = qc // L
            l = qc - bb * L
            vals = plsc.load_gather(idx2_v, [bb, l], mask=msk)
            plsc.store_scatter(idx_v, [qc], vals, mask=msk)
            return carry2

        lax.fori_loop(0, (CL + LANES - 1) // LANES, flat_body, 0, unroll=2)
        pltpu.async_copy(comb_hbm.at[idx_v], comb_v, sem)

    def wait_gather(idx_v, comb_v, sem):
        pltpu.make_async_copy(comb_hbm.at[idx_v], comb_v, sem).wait()

    def drain_write(c):
        b0 = base_w + c * CB
        pltpu.make_async_copy(asm_v, out_hbm.at[pl.ds(b0, CB)], sem_w).wait()

    def assemble_and_write(c, comb_v):
        def row_body(l, carry2):
            for bb in range(CB):
                for j in range(CMAIN // LANES):
                    sl = pl.ds(j * LANES, LANES)
                    asm_v[bb, l, sl] = comb_v[bb * L + l, sl]
            return carry2

        lax.fori_loop(0, L, row_body, 0, unroll=2)

        def tail_body(v, carry2):
            q = jax.lax.iota(jnp.int32, LANES) + v * LANES
            r = q // CTAIL
            col = q - r * CTAIL
            bb = r // L
            l = r - bb * L
            vals = plsc.load_gather(comb_v, [r, col + CMAIN])
            plsc.store_scatter(asm_v, [bb, l, col + CMAIN], vals)
            return carry2

        lax.fori_loop(0, CL * CTAIL // LANES, tail_body, 0, unroll=2)
        b0 = base_w + c * CB
        pltpu.async_copy(asm_v, out_hbm.at[pl.ds(b0, CB)], sem_w)

    start_gather(0, idx2_a, idx_a, comb_a, sem_a)

    def body2(k, carry):
        c0 = 2 * k
        start_gather(c0 + 1, idx2_b, idx_b, comb_b, sem_b)
        wait_gather(idx_a, comb_a, sem_a)

        @pl.when(k > 0)
        def _():
            drain_write(c0 - 1)

        assemble_and_write(c0, comb_a)

        @pl.when(k < n2 - 1)
        def _():
            start_gather(c0 + 2, idx2_a, idx_a, comb_a, sem_a)

        wait_gather(idx_b, comb_b, sem_b)
        drain_write(c0)
        assemble_and_write(c0 + 1, comb_b)
        return carry

    lax.fori_loop(0, n2, body2, 0)
    drain_write(n_chunks - 1)


def _comb_body(g_ref, t_ref, o_ref):
    o_ref[:, :DIM] = jnp.tanh(t_ref[...])
    o_ref[:, DIM:OUT_DIM] = g_ref[...]
    o_ref[:, OUT_DIM:] = jnp.zeros(
        (o_ref.shape[0], COMB_DIM - OUT_DIM), jnp.float32)


def _build_comb(glove_table, table):
    v = glove_table.shape[0]
    rows = 2000
    return pl.pallas_call(
        _comb_body,
        grid=(v // rows,),
        in_specs=[pl.BlockSpec((rows, GLOVE_DIM), lambda i: (i, 0)),
                  pl.BlockSpec((rows, DIM), lambda i: (i, 0))],
        out_specs=pl.BlockSpec((rows, COMB_DIM), lambda i: (i, 0)),
        out_shape=jax.ShapeDtypeStruct((v, COMB_DIM), jnp.float32),
    )(glove_table, table)


@jax.jit
def _glove_fused(x, glove_table, table):
    nb = x.shape[0]
    comb = _build_comb(glove_table, table)
    mesh = plsc.VectorSubcoreMesh(
        core_axis_name="c", subcore_axis_name="s",
        num_cores=NC, num_subcores=NS)
    return pl.kernel(
        _sc_body,
        out_type=jax.ShapeDtypeStruct((nb, L, OUT_DIM), jnp.float32),
        mesh=mesh,
        scratch_types=[
            pltpu.VMEM((CB, L), jnp.int32),
            pltpu.VMEM((CB, L), jnp.int32),
            pltpu.VMEM((CL,), jnp.int32),
            pltpu.VMEM((CL,), jnp.int32),
            pltpu.VMEM((CL, COMB_DIM), jnp.float32),
            pltpu.VMEM((CL, COMB_DIM), jnp.float32),
            pltpu.VMEM((CB, L, OUT_DIM), jnp.float32),
            pltpu.SemaphoreType.DMA,
            pltpu.SemaphoreType.DMA,
            pltpu.SemaphoreType.DMA,
        ],
        compiler_params=pltpu.CompilerParams(
            use_tc_tiling_on_sc=False, needs_layout_passes=False),
    )(x, comb)


def kernel(x, glove_table, table):
    return _glove_fused(x, glove_table, table)
